# Initial kernel scaffold; baseline (speedup 1.0000x reference)
import jax
import jax.numpy as jnp
from jax.experimental import pallas as pl


def kernel(x, table):
    raise NotImplementedError("write your pallas kernel here")


# speedup vs baseline: 3.8134x; 3.8134x over previous
import functools

import jax
import jax.numpy as jnp
from jax import lax
from jax.experimental import pallas as pl
from jax.experimental.pallas import tpu as pltpu
from jax.experimental.pallas import tpu_sc as plsc

_D = 64
_NW = 32
_CHUNK = 128


@functools.cache
def _make_gather(n_rows: int, n_var: int):
    b_per_w = n_rows // _NW
    n_chunks = b_per_w // _CHUNK
    mesh = plsc.VectorSubcoreMesh(core_axis_name="c", subcore_axis_name="s")

    @functools.partial(
        pl.kernel,
        mesh=mesh,
        out_type=jax.ShapeDtypeStruct((n_rows, _D), jnp.float32),
        scratch_types=[
            pltpu.VMEM((n_chunks, _CHUNK), jnp.int32),
            pltpu.VMEM((_CHUNK, _D), jnp.float32),
            pltpu.SemaphoreType.DMA,
        ],
        compiler_params=pltpu.CompilerParams(use_tc_tiling_on_sc=False),
    )
    def gather_kernel(idx_hbm, table_hbm, out_hbm, idx_v, rows_v, sem):
        wid = lax.axis_index("s") * 2 + lax.axis_index("c")
        base = wid * b_per_w
        pltpu.sync_copy(idx_hbm.at[wid], idx_v)

        def body(c, carry):
            pltpu.async_copy(table_hbm.at[idx_v.at[c]], rows_v, sem).wait()
            pltpu.sync_copy(rows_v, out_hbm.at[pl.ds(base + c * _CHUNK, _CHUNK)])
            return carry

        lax.fori_loop(0, n_chunks, body, 0)

    return gather_kernel


def kernel(x, table):
    n_rows = x.shape[0] * x.shape[1]
    idx = x.astype(jnp.int32).reshape(_NW, n_rows // (_NW * _CHUNK), _CHUNK)
    out = _make_gather(n_rows, table.shape[0])(idx, table)
    return out.reshape(x.shape + (_D,))


# speedup vs baseline: 4.4287x; 1.1613x over previous
import functools

import jax
import jax.numpy as jnp
from jax import lax
from jax.experimental import pallas as pl
from jax.experimental.pallas import tpu as pltpu
from jax.experimental.pallas import tpu_sc as plsc

_D = 64
_NW = 32
_CHUNK = 128


_K = 4
_GROUP = _K * _CHUNK


@functools.cache
def _make_gather(n_rows: int, n_var: int):
    b_per_w = n_rows // _NW
    n_chunks = b_per_w // _CHUNK
    n_groups = n_chunks // _K
    assert n_chunks % _K == 0 and n_groups % 2 == 0
    mesh = plsc.VectorSubcoreMesh(core_axis_name="c", subcore_axis_name="s")

    @functools.partial(
        pl.kernel,
        mesh=mesh,
        out_type=jax.ShapeDtypeStruct((n_rows, _D), jnp.float32),
        scratch_types=[
            pltpu.VMEM((n_chunks, _CHUNK), jnp.int32),
            pltpu.VMEM((2, _GROUP, _D), jnp.float32),
            pltpu.SemaphoreType.DMA,
            pltpu.SemaphoreType.DMA,
            pltpu.SemaphoreType.DMA,
        ],
        compiler_params=pltpu.CompilerParams(use_tc_tiling_on_sc=False),
    )
    def gather_kernel(idx_hbm, table_hbm, out_hbm, idx_v, rows_v, gsem, wsem0, wsem1):
        wid = lax.axis_index("s") * 2 + lax.axis_index("c")
        base = wid * b_per_w
        pltpu.sync_copy(idx_hbm.at[wid], idx_v)

        def fire_gathers(g, h):
            for k in range(_K):
                pltpu.async_copy(
                    table_hbm.at[idx_v.at[g * _K + k]],
                    rows_v.at[h].at[pl.ds(k * _CHUNK, _CHUNK)], gsem)

        def drain_gathers(g, h):
            for k in range(_K):
                pltpu.make_async_copy(
                    table_hbm.at[idx_v.at[g * _K + k]],
                    rows_v.at[h].at[pl.ds(k * _CHUNK, _CHUNK)], gsem).wait()

        def write_copy(g, h, wsem):
            return pltpu.make_async_copy(
                rows_v.at[h], out_hbm.at[pl.ds(base + g * _GROUP, _GROUP)], wsem)

        def do_group(g, h, wsem):
            fire_gathers(g, h)
            drain_gathers(g, h)
            write_copy(g, h, wsem).start()

        do_group(0, 0, wsem0)
        do_group(1, 1, wsem1)

        def body(i, carry):
            g0 = 2 * i
            write_copy(g0 - 2, 0, wsem0).wait()
            do_group(g0, 0, wsem0)
            g1 = 2 * i + 1
            write_copy(g1 - 2, 1, wsem1).wait()
            do_group(g1, 1, wsem1)
            return carry

        lax.fori_loop(1, n_groups // 2, body, 0)
        write_copy(n_groups - 2, 0, wsem0).wait()
        write_copy(n_groups - 1, 1, wsem1).wait()

    return gather_kernel


def kernel(x, table):
    n_rows = x.shape[0] * x.shape[1]
    idx = x.astype(jnp.int32).reshape(_NW, n_rows // (_NW * _CHUNK), _CHUNK)
    out = _make_gather(n_rows, table.shape[0])(idx, table)
    return out.reshape(x.shape + (_D,))
